# baseline (speedup 1.0000x reference)
import math

import jax
import jax.numpy as jnp
from jax import lax
from jax.experimental import pallas as pl
from jax.experimental.pallas import tpu as pltpu
from jax.experimental.pallas import tpu_sc as plsc

_EM = math.exp(-1e-6 - 1.0)

_P = 1024
_M = 64
_NC = 2
_NS = 16
_NW = _NC * _NS
_ROWS = _P // _NW
_L = 16
_BR = 4
_NB = _ROWS // _BR


def _sc_body(nei_hbm, cnt_hbm, nei_v0, nei_v1, cnt_v, sem0, sem1):
    wid = lax.axis_index("s") * _NC + lax.axis_index("c")
    base = wid * _ROWS
    bufs = (nei_v0, nei_v1)
    sems = (sem0, sem1)

    zero = jnp.zeros((_L,), jnp.float32)
    one = jnp.full((_L,), 1.0, jnp.float32)

    pltpu.async_copy(nei_hbm.at[pl.ds(base * _P, _BR * _P)], nei_v0,
                     sem0).wait()
    for b in range(_NB):
        buf = bufs[b % 2]
        if b + 1 < _NB:
            nxt = pltpu.async_copy(
                nei_hbm.at[pl.ds((base + (b + 1) * _BR) * _P, _BR * _P)],
                bufs[(b + 1) % 2], sems[(b + 1) % 2])
        for rr in range(_BR):
            def chunk_body(jc, cv, buf=buf, rr=rr):
                nv = buf[pl.ds(rr * _P + jc * _L, _L)]
                return cv + jnp.where(nv > 0, one, zero)

            cv = lax.fori_loop(0, _P // _L, chunk_body, zero)
            cnt_v[pl.ds((b * _BR + rr) * _L, _L)] = cv
        if b + 1 < _NB:
            nxt.wait()

    pltpu.sync_copy(cnt_v, cnt_hbm.at[pl.ds(base * _L, _ROWS * _L)])


_sc_call = pl.kernel(
    _sc_body,
    out_type=jax.ShapeDtypeStruct((_P * _L,), jnp.float32),
    mesh=plsc.VectorSubcoreMesh(core_axis_name="c", subcore_axis_name="s"),
    scratch_types=[
        pltpu.VMEM((_BR * _P,), jnp.int32),
        pltpu.VMEM((_BR * _P,), jnp.int32),
        pltpu.VMEM((_ROWS * _L,), jnp.float32),
        pltpu.SemaphoreType.DMA,
        pltpu.SemaphoreType.DMA,
    ],
)


def _tc_body(hs_ref, nei_ref, out_ref):
    mf = (nei_ref[...] > 0).astype(jnp.float32)
    out_ref[...] = jnp.dot(mf, hs_ref[...],
                           preferred_element_type=jnp.float32)


def _ep_body(acc_ref, cnt_ref, hs_ref, out_ref):
    k = jnp.sum(cnt_ref[...], axis=1, keepdims=True)
    scale = 1.0 / (k + (_P - k) * _EM)
    has = jnp.any(k > 0.0)
    out_ref[...] = jnp.where(has, acc_ref[...] * scale, hs_ref[...])


def kernel(hidden_state, corr_index, nei_index):
    del corr_index
    lane_cnt = _sc_call(nei_index.reshape(-1))
    acc = pl.pallas_call(
        _tc_body,
        out_shape=jax.ShapeDtypeStruct((_P, _M), jnp.float32),
    )(hidden_state, nei_index)
    return pl.pallas_call(
        _ep_body,
        out_shape=jax.ShapeDtypeStruct((_P, _M), jnp.float32),
    )(acc, lane_cnt.reshape(_P, _L), hidden_state)


# speedup vs baseline: 2.7562x; 2.7562x over previous
import math

import jax
import jax.numpy as jnp
from jax.experimental import pallas as pl

_EM = math.exp(-1e-6 - 1.0)

_BLK = 128


def _mm_body(hs_ref, nei_ref, out_ref, k_ref):
    p = jnp.float32(nei_ref.shape[1])
    mf = (nei_ref[...] > 0).astype(jnp.float32)
    k = jnp.sum(mf, axis=1, keepdims=True)
    scale = 1.0 / (k + (p - k) * _EM)
    acc = jnp.dot(mf, hs_ref[...], preferred_element_type=jnp.float32)
    out_ref[...] = scale * acc
    k_ref[...] = k


def _ep_body(acc_ref, k_ref, hs_ref, out_ref):
    has = jnp.any(k_ref[...] > 0.0)
    out_ref[...] = jnp.where(has, acc_ref[...], hs_ref[...])


def kernel(hidden_state, corr_index, nei_index):
    del corr_index
    ped_num, m_dim = hidden_state.shape
    grid = (ped_num // _BLK,)
    acc, k = pl.pallas_call(
        _mm_body,
        grid=grid,
        in_specs=[
            pl.BlockSpec((ped_num, m_dim), lambda g: (0, 0)),
            pl.BlockSpec((_BLK, ped_num), lambda g: (g, 0)),
        ],
        out_specs=[
            pl.BlockSpec((_BLK, m_dim), lambda g: (g, 0)),
            pl.BlockSpec((_BLK, 1), lambda g: (g, 0)),
        ],
        out_shape=[
            jax.ShapeDtypeStruct((ped_num, m_dim), jnp.float32),
            jax.ShapeDtypeStruct((ped_num, 1), jnp.float32),
        ],
    )(hidden_state, nei_index)
    return pl.pallas_call(
        _ep_body,
        out_shape=jax.ShapeDtypeStruct((ped_num, m_dim), jnp.float32),
    )(acc, k, hidden_state)


# speedup vs baseline: 3.2279x; 1.1711x over previous
import math

import jax
import jax.numpy as jnp
from jax.experimental import pallas as pl

_EM = math.exp(-1e-6 - 1.0)

_BLK = 256


def _mm_body(hs_ref, nei_ref, out_ref, k_ref):
    p = jnp.float32(nei_ref.shape[1])
    mf = (nei_ref[...] > 0).astype(jnp.float32)
    k = jnp.sum(mf, axis=1, keepdims=True)
    scale = 1.0 / (k + (p - k) * _EM)
    acc = jnp.dot(mf, hs_ref[...], preferred_element_type=jnp.float32)
    out_ref[...] = scale * acc
    k_ref[...] = k


def _ep_body(acc_ref, k_ref, hs_ref, out_ref):
    has = jnp.any(k_ref[...] > 0.0)
    out_ref[...] = jnp.where(has, acc_ref[...], hs_ref[...])


def kernel(hidden_state, corr_index, nei_index):
    del corr_index
    ped_num, m_dim = hidden_state.shape
    grid = (ped_num // _BLK,)
    acc, k = pl.pallas_call(
        _mm_body,
        grid=grid,
        in_specs=[
            pl.BlockSpec((ped_num, m_dim), lambda g: (0, 0)),
            pl.BlockSpec((_BLK, ped_num), lambda g: (g, 0)),
        ],
        out_specs=[
            pl.BlockSpec((_BLK, m_dim), lambda g: (g, 0)),
            pl.BlockSpec((_BLK, 1), lambda g: (g, 0)),
        ],
        out_shape=[
            jax.ShapeDtypeStruct((ped_num, m_dim), jnp.float32),
            jax.ShapeDtypeStruct((ped_num, 1), jnp.float32),
        ],
    )(hidden_state, nei_index)
    return jnp.where(jnp.any(k > 0.0), acc, hidden_state)


# speedup vs baseline: 4.5695x; 1.4156x over previous
import math

import jax
import jax.numpy as jnp
from jax.experimental import pallas as pl

_EM = math.exp(-1e-6 - 1.0)


def _body(hs_ref, nei_ref, out_ref):
    p = jnp.float32(nei_ref.shape[1])
    m = hs_ref.shape[1]
    mask = nei_ref[...] > 0
    mf = mask.astype(jnp.float32)
    aug = jnp.concatenate(
        [hs_ref[...], jnp.ones((hs_ref.shape[0], 1), jnp.float32)], axis=1)
    acck = jnp.dot(mf, aug, preferred_element_type=jnp.float32)
    k = acck[:, m:]
    scale = 1.0 / (k + (p - k) * _EM)
    has = jnp.any(mask)
    out_ref[...] = jnp.where(has, scale * acck[:, :m], hs_ref[...])


def kernel(hidden_state, corr_index, nei_index):
    del corr_index
    ped_num, m_dim = hidden_state.shape
    return pl.pallas_call(
        _body,
        out_shape=jax.ShapeDtypeStruct((ped_num, m_dim), jnp.float32),
    )(hidden_state, nei_index)
